# baseline (speedup 1.0000x reference)
import functools

import jax
import jax.numpy as jnp
import numpy as np
from jax import lax
from jax.experimental import pallas as pl
from jax.experimental.pallas import tpu as pltpu
from jax.experimental.pallas import tpu_sc as plsc

C = 384
G = 8
P = 9
NCORN = 4
BS = 4
H = 32
W = 32
HW = H * W
ROWS = BS * HW
LANE = 16
NSLAB = C // LANE
HALF = HW // 2
NTASK = BS * NSLAB * 2
NWORK = 32
TASKS_PER_W = NTASK // NWORK

_ROWBLK = 512


_NT = (((1,), (1,)), ((), ()))


def _prep_body(x_ref, wv_ref, bv_ref, ww_ref, bw72_ref, sm_cols_ref, seg_ref,
               wo_ref, bo18_ref, offx_ref, offy_ref, axy_ref,
               v_ref, idx_ref, bw_ref, sm_ref):
    x = x_ref[...]
    f32 = jnp.float32
    v_ref[...] = (lax.dot_general(x, wv_ref[...], _NT,
                                  preferred_element_type=f32, precision=lax.Precision.HIGHEST) + bv_ref[...])
    wr = (lax.dot_general(x, ww_ref[...], _NT, preferred_element_type=f32, precision=lax.Precision.HIGHEST)
          + bw72_ref[...])
    e = jnp.exp(jnp.dot(wr, sm_cols_ref[...], preferred_element_type=f32, precision=lax.Precision.HIGHEST))
    z = jnp.dot(e, seg_ref[...], preferred_element_type=f32, precision=lax.Precision.HIGHEST)
    sm_ref[...] = e / z
    off = (lax.dot_general(x, wo_ref[...], _NT, preferred_element_type=f32, precision=lax.Precision.HIGHEST)
           + bo18_ref[...])
    xp = (jnp.dot(off, offx_ref[...], preferred_element_type=f32, precision=lax.Precision.HIGHEST)
          + axy_ref[:, 0:1])
    yp = (jnp.dot(off, offy_ref[...], preferred_element_type=f32, precision=lax.Precision.HIGHEST)
          + axy_ref[:, 1:2])
    lane = lax.broadcasted_iota(jnp.int32, xp.shape, 1)
    cidx = lane >> 4
    is_x1 = (cidx & 1) == 1
    is_y1 = cidx >= 2
    one = jnp.float32(1.0)
    xf = jnp.floor(xp)
    dx = xp - xf
    xc = xf + jnp.where(is_x1, one, 0.0)
    wxc = jnp.where(is_x1, dx, one - dx)
    yf = jnp.floor(yp)
    dy = yp - yf
    yc = yf + jnp.where(is_y1, one, 0.0)
    wyc = jnp.where(is_y1, dy, one - dy)
    valid = ((xc >= 0) & (xc <= W - 1) & (yc >= 0) & (yc <= H - 1))
    bw = wxc * wyc * jnp.where(valid, one, 0.0)
    xi = jnp.clip(xc, 0, W - 1).astype(jnp.int32)
    yi = jnp.clip(yc, 0, H - 1).astype(jnp.int32)
    idx = yi * W + xi
    idx_ref[...] = idx[:, 0:64]
    bw_ref[...] = bw[:, 0:64]


def _prep(x, wv, bv, ww, bw72, sm_cols, seg, wo, bo18, offx, offy, axy):
    nblk = ROWS // _ROWBLK
    row_spec = lambda nc: pl.BlockSpec((_ROWBLK, nc), lambda i: (i, 0))
    full = lambda a: pl.BlockSpec(a.shape, lambda i: (0,) * a.ndim)
    return pl.pallas_call(
        _prep_body,
        grid=(nblk,),
        in_specs=[row_spec(C), full(wv), full(bv), full(ww), full(bw72),
                  full(sm_cols), full(seg), full(wo), full(bo18),
                  full(offx), full(offy), row_spec(2)],
        out_specs=[row_spec(C), row_spec(64), row_spec(64), row_spec(128)],
        out_shape=[
            jax.ShapeDtypeStruct((ROWS, C), jnp.float32),
            jax.ShapeDtypeStruct((ROWS, 64), jnp.int32),
            jax.ShapeDtypeStruct((ROWS, 64), jnp.float32),
            jax.ShapeDtypeStruct((ROWS, 128), jnp.float32),
        ],
    )(x, wv, bv, ww, bw72, sm_cols, seg, wo, bo18, offx, offy, axy)


def _sc_body(v_hbm, idx_hbm, bw_hbm, sm_hbm, out_hbm,
             vslab, idx_v, bw_v, sm_v, out_v):
    wid = lax.axis_index("s") * 2 + lax.axis_index("c")
    iota16 = lax.iota(jnp.int32, LANE)
    for r in range(TASKS_PER_W):
        t = wid * TASKS_PER_W + r
        b = t // (NSLAB * 2)
        rem = t % (NSLAB * 2)
        slab = rem // 2
        half = rem % 2
        grp = slab // (NSLAB // G)
        row0 = b * HW + half * HALF
        col0 = slab * LANE
        pltpu.sync_copy(v_hbm.at[pl.ds(b * HW, HW), pl.ds(col0, LANE)],
                        vslab)
        pltpu.sync_copy(idx_hbm.at[pl.ds(row0, HALF)], idx_v)
        pltpu.sync_copy(bw_hbm.at[pl.ds(row0, HALF)], bw_v)
        pltpu.sync_copy(sm_hbm.at[pl.ds(row0, HALF), pl.ds(grp * LANE, LANE)],
                        sm_v)
        zeros16 = jnp.zeros((LANE,), jnp.int32)

        @plsc.parallel_loop(0, HALF, unroll=4)
        def loc(i):
            smr = sm_v[i]
            idxr = [idx_v[i, pl.ds(c * LANE, LANE)] for c in range(NCORN)]
            bwr = [bw_v[i, pl.ds(c * LANE, LANE)] for c in range(NCORN)]
            acc = jnp.zeros((LANE,), jnp.float32)
            for p in range(P):
                smv = smr[p]
                ptmp = jnp.zeros((LANE,), jnp.float32)
                for c in range(NCORN):
                    rid = idxr[c][p]
                    bwv = bwr[c][p]
                    vals = vslab[rid]
                    ptmp = ptmp + vals * bwv
                acc = acc + ptmp * smv
            out_v[i] = acc
        pltpu.sync_copy(out_v,
                        out_hbm.at[pl.ds(row0, HALF), pl.ds(col0, LANE)])


@functools.lru_cache(maxsize=1)
def _sc_aggregate_fn():
    return pl.kernel(
        _sc_body,
        out_type=jax.ShapeDtypeStruct((ROWS, C), jnp.float32),
        mesh=plsc.VectorSubcoreMesh(core_axis_name="c",
                                    subcore_axis_name="s"),
        compiler_params=pltpu.CompilerParams(use_tc_tiling_on_sc=False,
                                             needs_layout_passes=False),
        scratch_types=[
            pltpu.VMEM((HW, LANE), jnp.float32),
            pltpu.VMEM((HALF, 64), jnp.int32),
            pltpu.VMEM((HALF, 64), jnp.float32),
            pltpu.VMEM((HALF, LANE), jnp.float32),
            pltpu.VMEM((HALF, LANE), jnp.float32),
        ],
    )


def _out_body(x_ref, w_ref, b_ref, o_ref):
    ot = (lax.dot_general(w_ref[...], x_ref[...], _NT,
                          preferred_element_type=jnp.float32,
                          precision=lax.Precision.HIGHEST)
          + b_ref[...])
    o_ref[...] = ot[None]


def _out_proj(x, w, b):
    nblk = ROWS // _ROWBLK
    per_b = HW // _ROWBLK
    return pl.pallas_call(
        _out_body,
        grid=(nblk,),
        in_specs=[pl.BlockSpec((_ROWBLK, C), lambda i: (i, 0)),
                  pl.BlockSpec((C, C), lambda i: (0, 0)),
                  pl.BlockSpec((C, 1), lambda i: (0, 0))],
        out_specs=pl.BlockSpec((1, C, _ROWBLK),
                               lambda i: (i // per_b, 0, i % per_b)),
        out_shape=jax.ShapeDtypeStruct((BS, C, HW), jnp.float32),
    )(x, w, b.reshape(C, 1))


def _lane_maps():
    sm_cols = np.zeros((G * P, 128), np.float32)
    for g in range(G):
        for p in range(P):
            sm_cols[p * G + g, g * LANE + p] = 1.0
    seg = np.zeros((128, 128), np.float32)
    for g in range(G):
        for p in range(P):
            for q in range(P):
                seg[g * LANE + p, g * LANE + q] = 1.0
    offx = np.zeros((2 * P, 128), np.float32)
    offy = np.zeros((2 * P, 128), np.float32)
    for c in range(NCORN):
        for p in range(P):
            offx[p * 2 + 0, c * LANE + p] = float(W)
            offy[p * 2 + 1, c * LANE + p] = float(H)
    return sm_cols, seg, offx, offy


_SM_COLS, _SEG, _OFFX, _OFFY = _lane_maps()


@jax.jit
def kernel(feats, anchor_points, W_value, b_value, W_weights, b_weights,
           W_offset, b_offset, W_out, b_out):
    x = feats.reshape(BS, C, HW).transpose(0, 2, 1).reshape(ROWS, C)

    axy = (anchor_points.reshape(ROWS, 2) * jnp.float32(W)
           - jnp.float32(0.5))

    v, idx, bw, sm = _prep(x, W_value, b_value.reshape(1, C),
                           W_weights, b_weights.reshape(1, G * P),
                           jnp.asarray(_SM_COLS), jnp.asarray(_SEG),
                           W_offset, b_offset.reshape(1, 2 * P),
                           jnp.asarray(_OFFX), jnp.asarray(_OFFY), axy)
    out_pre = _sc_aggregate_fn()(v, idx, bw, sm)
    out = _out_proj(out_pre, W_out, b_out)
    return out.reshape(BS, C, H, W)


# speedup vs baseline: 1.0776x; 1.0776x over previous
import functools

import jax
import jax.numpy as jnp
import numpy as np
from jax import lax
from jax.experimental import pallas as pl
from jax.experimental.pallas import tpu as pltpu
from jax.experimental.pallas import tpu_sc as plsc

C = 384
G = 8
P = 9
NCORN = 4
BS = 4
H = 32
W = 32
HW = H * W
ROWS = BS * HW
LANE = 16
NSLAB = C // LANE
HALF = HW // 2
NTASK = BS * NSLAB * 2
NWORK = 32
TASKS_PER_W = NTASK // NWORK

_ROWBLK = 512


_NT = (((1,), (1,)), ((), ()))


def _prep_body(x_ref, wv_ref, bv_ref, wsm_ref, bsm_ref, seg_ref,
               wx_ref, wy_ref, bxy_ref, axy_ref,
               v_ref, idx_ref, bw_ref, sm_ref):
    x = x_ref[...]
    f32 = jnp.float32
    v_ref[...] = (jnp.dot(x, wv_ref[...], preferred_element_type=f32)
                  + bv_ref[...])
    e = jnp.exp(jnp.dot(x, wsm_ref[...], preferred_element_type=f32)
                + bsm_ref[...])
    z = jnp.dot(e, seg_ref[...], preferred_element_type=f32)
    sm_ref[...] = e / z
    xp = (jnp.dot(x, wx_ref[...], preferred_element_type=f32)
          + bxy_ref[0:1, :] + axy_ref[:, 0:1])
    yp = (jnp.dot(x, wy_ref[...], preferred_element_type=f32)
          + bxy_ref[1:2, :] + axy_ref[:, 1:2])
    lane = lax.broadcasted_iota(jnp.int32, xp.shape, 1)
    cidx = lane >> 4
    is_x1 = (cidx & 1) == 1
    is_y1 = cidx >= 2
    one = jnp.float32(1.0)
    xf = jnp.floor(xp)
    dx = xp - xf
    xc = xf + jnp.where(is_x1, one, 0.0)
    wxc = jnp.where(is_x1, dx, one - dx)
    yf = jnp.floor(yp)
    dy = yp - yf
    yc = yf + jnp.where(is_y1, one, 0.0)
    wyc = jnp.where(is_y1, dy, one - dy)
    valid = ((xc >= 0) & (xc <= W - 1) & (yc >= 0) & (yc <= H - 1))
    bw = wxc * wyc * jnp.where(valid, one, 0.0)
    xi = jnp.clip(xc, 0, W - 1).astype(jnp.int32)
    yi = jnp.clip(yc, 0, H - 1).astype(jnp.int32)
    idx = yi * W + xi
    idx_ref[...] = idx[:, 0:64]
    bw_ref[...] = bw[:, 0:64]


def _prep(x, wv, bv, wsm, bsm, seg, wx, wy, bxy, axy):
    nblk = ROWS // _ROWBLK
    row_spec = lambda nc: pl.BlockSpec((_ROWBLK, nc), lambda i: (i, 0))
    full = lambda a: pl.BlockSpec(a.shape, lambda i: (0,) * a.ndim)
    return pl.pallas_call(
        _prep_body,
        grid=(nblk,),
        in_specs=[row_spec(C), full(wv), full(bv), full(wsm), full(bsm),
                  full(seg), full(wx), full(wy), full(bxy), row_spec(2)],
        out_specs=[row_spec(C), row_spec(64), row_spec(64), row_spec(128)],
        out_shape=[
            jax.ShapeDtypeStruct((ROWS, C), jnp.float32),
            jax.ShapeDtypeStruct((ROWS, 64), jnp.int32),
            jax.ShapeDtypeStruct((ROWS, 64), jnp.float32),
            jax.ShapeDtypeStruct((ROWS, 128), jnp.float32),
        ],
    )(x, wv, bv, wsm, bsm, seg, wx, wy, bxy, axy)


def _sc_body(v_hbm, idx_hbm, bw_hbm, sm_hbm, out_hbm,
             vslab, idx_v, bw_v, sm_v, out_v):
    wid = lax.axis_index("s") * 2 + lax.axis_index("c")
    iota16 = lax.iota(jnp.int32, LANE)
    for r in range(TASKS_PER_W):
        t = wid * TASKS_PER_W + r
        b = t // (NSLAB * 2)
        rem = t % (NSLAB * 2)
        slab = rem // 2
        half = rem % 2
        grp = slab // (NSLAB // G)
        row0 = b * HW + half * HALF
        col0 = slab * LANE
        pltpu.sync_copy(v_hbm.at[pl.ds(b * HW, HW), pl.ds(col0, LANE)],
                        vslab)
        pltpu.sync_copy(idx_hbm.at[pl.ds(row0, HALF)], idx_v)
        pltpu.sync_copy(bw_hbm.at[pl.ds(row0, HALF)], bw_v)
        pltpu.sync_copy(sm_hbm.at[pl.ds(row0, HALF), pl.ds(grp * LANE, LANE)],
                        sm_v)
        zeros16 = jnp.zeros((LANE,), jnp.int32)

        @plsc.parallel_loop(0, HALF, unroll=4)
        def loc(i):
            smr = sm_v[i]
            idxr = [idx_v[i, pl.ds(c * LANE, LANE)] for c in range(NCORN)]
            bwr = [bw_v[i, pl.ds(c * LANE, LANE)] for c in range(NCORN)]
            acc = jnp.zeros((LANE,), jnp.float32)
            for p in range(P):
                smv = smr[p]
                ptmp = jnp.zeros((LANE,), jnp.float32)
                for c in range(NCORN):
                    rid = idxr[c][p]
                    bwv = bwr[c][p]
                    vals = vslab[rid]
                    ptmp = ptmp + vals * bwv
                acc = acc + ptmp * smv
            out_v[i] = acc
        pltpu.sync_copy(out_v,
                        out_hbm.at[pl.ds(row0, HALF), pl.ds(col0, LANE)])


@functools.lru_cache(maxsize=1)
def _sc_aggregate_fn():
    return pl.kernel(
        _sc_body,
        out_type=jax.ShapeDtypeStruct((ROWS, C), jnp.float32),
        mesh=plsc.VectorSubcoreMesh(core_axis_name="c",
                                    subcore_axis_name="s"),
        compiler_params=pltpu.CompilerParams(use_tc_tiling_on_sc=False,
                                             needs_layout_passes=False),
        scratch_types=[
            pltpu.VMEM((HW, LANE), jnp.float32),
            pltpu.VMEM((HALF, 64), jnp.int32),
            pltpu.VMEM((HALF, 64), jnp.float32),
            pltpu.VMEM((HALF, LANE), jnp.float32),
            pltpu.VMEM((HALF, LANE), jnp.float32),
        ],
    )


def _out_body(x_ref, w_ref, b_ref, o_ref):
    ot = (lax.dot_general(w_ref[...], x_ref[...], _NT,
                          preferred_element_type=jnp.float32)
          + b_ref[...])
    o_ref[...] = ot[None]


def _out_proj(x, w, b):
    nblk = ROWS // _ROWBLK
    per_b = HW // _ROWBLK
    return pl.pallas_call(
        _out_body,
        grid=(nblk,),
        in_specs=[pl.BlockSpec((_ROWBLK, C), lambda i: (i, 0)),
                  pl.BlockSpec((C, C), lambda i: (0, 0)),
                  pl.BlockSpec((C, 1), lambda i: (0, 0))],
        out_specs=pl.BlockSpec((1, C, _ROWBLK),
                               lambda i: (i // per_b, 0, i % per_b)),
        out_shape=jax.ShapeDtypeStruct((BS, C, HW), jnp.float32),
    )(x, w, b.reshape(C, 1))


def _lane_maps():
    sm_cols = np.zeros((G * P, 128), np.float32)
    for g in range(G):
        for p in range(P):
            sm_cols[p * G + g, g * LANE + p] = 1.0
    seg = np.zeros((128, 128), np.float32)
    for g in range(G):
        for p in range(P):
            for q in range(P):
                seg[g * LANE + p, g * LANE + q] = 1.0
    offx = np.zeros((2 * P, 128), np.float32)
    offy = np.zeros((2 * P, 128), np.float32)
    for c in range(NCORN):
        for p in range(P):
            offx[p * 2 + 0, c * LANE + p] = float(W)
            offy[p * 2 + 1, c * LANE + p] = float(H)
    return sm_cols, seg, offx, offy


_SM_COLS, _SEG, _OFFX, _OFFY = _lane_maps()


@jax.jit
def kernel(feats, anchor_points, W_value, b_value, W_weights, b_weights,
           W_offset, b_offset, W_out, b_out):
    x = feats.reshape(BS, C, HW).transpose(0, 2, 1).reshape(ROWS, C)

    axy = (anchor_points.reshape(ROWS, 2) * jnp.float32(W)
           - jnp.float32(0.5))

    sm_cols = jnp.asarray(_SM_COLS)
    wsm = W_weights.T @ sm_cols
    bsm = (b_weights @ sm_cols).reshape(1, 128)
    wx = W_offset.T @ jnp.asarray(_OFFX)
    wy = W_offset.T @ jnp.asarray(_OFFY)
    bxy = jnp.stack([b_offset @ jnp.asarray(_OFFX),
                     b_offset @ jnp.asarray(_OFFY)])
    v, idx, bw, sm = _prep(x, W_value.T, b_value.reshape(1, C),
                           wsm, bsm, jnp.asarray(_SEG), wx, wy, bxy, axy)
    out_pre = _sc_aggregate_fn()(v, idx, bw, sm)
    out = _out_proj(out_pre, W_out, b_out)
    return out.reshape(BS, C, H, W)


# speedup vs baseline: 1.1647x; 1.0808x over previous
import functools

import jax
import jax.numpy as jnp
import numpy as np
from jax import lax
from jax.experimental import pallas as pl
from jax.experimental.pallas import tpu as pltpu
from jax.experimental.pallas import tpu_sc as plsc

C = 384
G = 8
P = 9
NCORN = 4
BS = 4
H = 32
W = 32
HW = H * W
ROWS = BS * HW
LANE = 16
NSLAB = C // LANE
HALF = HW // 2
NTASK = BS * NSLAB * 2
NWORK = 32
TASKS_PER_W = NTASK // NWORK

_ROWBLK = 512


_NT = (((1,), (1,)), ((), ()))


def _prep_body(x_ref, wv_ref, bv_ref, wsm_ref, bsm_ref, seg_ref,
               wx_ref, wy_ref, bxy_ref, axy_ref,
               v_ref, idx_ref, coef_ref):
    x = x_ref[...]
    f32 = jnp.float32
    v_ref[...] = (jnp.dot(x, wv_ref[...], preferred_element_type=f32)
                  + bv_ref[...])
    e = jnp.exp(jnp.dot(x, wsm_ref[...], preferred_element_type=f32)
                + bsm_ref[...])
    z = jnp.dot(e, seg_ref[...], preferred_element_type=f32)
    sm = e / z
    xp = (jnp.dot(x, wx_ref[...], preferred_element_type=f32)
          + bxy_ref[0:1, :] + axy_ref[:, 0:1])
    yp = (jnp.dot(x, wy_ref[...], preferred_element_type=f32)
          + bxy_ref[1:2, :] + axy_ref[:, 1:2])
    lane = lax.broadcasted_iota(jnp.int32, xp.shape, 1)
    cidx = lane >> 4
    is_x1 = (cidx & 1) == 1
    is_y1 = cidx >= 2
    one = jnp.float32(1.0)
    xf = jnp.floor(xp)
    dx = xp - xf
    xc = xf + jnp.where(is_x1, one, 0.0)
    wxc = jnp.where(is_x1, dx, one - dx)
    yf = jnp.floor(yp)
    dy = yp - yf
    yc = yf + jnp.where(is_y1, one, 0.0)
    wyc = jnp.where(is_y1, dy, one - dy)
    valid = ((xc >= 0) & (xc <= W - 1) & (yc >= 0) & (yc <= H - 1))
    bw = wxc * wyc * jnp.where(valid, one, 0.0)
    xi = jnp.clip(xc, 0, W - 1).astype(jnp.int32)
    yi = jnp.clip(yc, 0, H - 1).astype(jnp.int32)
    idx = yi * W + xi
    idx_ref[...] = idx[:, 0:64]
    bw64 = bw[:, 0:64]
    parts = []
    for g in range(G):
        smg = sm[:, g * LANE:(g + 1) * LANE]
        smrep = jnp.concatenate([smg, smg, smg, smg], axis=1)
        parts.append(bw64 * smrep)
    coef_ref[...] = jnp.concatenate(parts, axis=1)


def _prep(x, wv, bv, wsm, bsm, seg, wx, wy, bxy, axy):
    nblk = ROWS // _ROWBLK
    row_spec = lambda nc: pl.BlockSpec((_ROWBLK, nc), lambda i: (i, 0))
    full = lambda a: pl.BlockSpec(a.shape, lambda i: (0,) * a.ndim)
    return pl.pallas_call(
        _prep_body,
        grid=(nblk,),
        in_specs=[row_spec(C), full(wv), full(bv), full(wsm), full(bsm),
                  full(seg), full(wx), full(wy), full(bxy), row_spec(2)],
        out_specs=[row_spec(C), row_spec(64), row_spec(G * 64)],
        out_shape=[
            jax.ShapeDtypeStruct((ROWS, C), jnp.float32),
            jax.ShapeDtypeStruct((ROWS, 64), jnp.int32),
            jax.ShapeDtypeStruct((ROWS, G * 64), jnp.float32),
        ],
    )(x, wv, bv, wsm, bsm, seg, wx, wy, bxy, axy)


def _sc_body(v_hbm, idx_hbm, coef_hbm, out_hbm,
             vslab, idx_v, coef_v, out_v):
    wid = lax.axis_index("s") * 2 + lax.axis_index("c")
    iota16 = lax.iota(jnp.int32, LANE)
    for r in range(TASKS_PER_W):
        t = wid * TASKS_PER_W + r
        b = t // (NSLAB * 2)
        rem = t % (NSLAB * 2)
        slab = rem // 2
        half = rem % 2
        grp = slab // (NSLAB // G)
        row0 = b * HW + half * HALF
        col0 = slab * LANE
        pltpu.sync_copy(v_hbm.at[pl.ds(b * HW, HW), pl.ds(col0, LANE)],
                        vslab)
        pltpu.sync_copy(idx_hbm.at[pl.ds(row0, HALF)], idx_v)
        pltpu.sync_copy(coef_hbm.at[pl.ds(row0, HALF), pl.ds(grp * 64, 64)],
                        coef_v)

        @plsc.parallel_loop(0, HALF, unroll=4)
        def loc(i):
            idxr = [idx_v[i, pl.ds(c * LANE, LANE)] for c in range(NCORN)]
            cfr = [coef_v[i, pl.ds(c * LANE, LANE)] for c in range(NCORN)]
            acc = jnp.zeros((LANE,), jnp.float32)
            for p in range(P):
                for c in range(NCORN):
                    rid = idxr[c][p]
                    cfv = cfr[c][p]
                    vals = vslab[rid]
                    acc = acc + vals * cfv
            out_v[i] = acc
        pltpu.sync_copy(out_v,
                        out_hbm.at[pl.ds(row0, HALF), pl.ds(col0, LANE)])


@functools.lru_cache(maxsize=1)
def _sc_aggregate_fn():
    return pl.kernel(
        _sc_body,
        out_type=jax.ShapeDtypeStruct((ROWS, C), jnp.float32),
        mesh=plsc.VectorSubcoreMesh(core_axis_name="c",
                                    subcore_axis_name="s"),
        compiler_params=pltpu.CompilerParams(use_tc_tiling_on_sc=False,
                                             needs_layout_passes=False),
        scratch_types=[
            pltpu.VMEM((HW, LANE), jnp.float32),
            pltpu.VMEM((HALF, 64), jnp.int32),
            pltpu.VMEM((HALF, 64), jnp.float32),
            pltpu.VMEM((HALF, LANE), jnp.float32),
        ],
    )


def _out_body(x_ref, w_ref, b_ref, o_ref):
    o_ref[...] = (jnp.dot(x_ref[...], w_ref[...],
                          preferred_element_type=jnp.float32) + b_ref[...])


def _out_proj(x, w_t, b):
    nblk = ROWS // _ROWBLK
    return pl.pallas_call(
        _out_body,
        grid=(nblk,),
        in_specs=[pl.BlockSpec((_ROWBLK, C), lambda i: (i, 0)),
                  pl.BlockSpec((C, C), lambda i: (0, 0)),
                  pl.BlockSpec((1, C), lambda i: (0, 0))],
        out_specs=pl.BlockSpec((_ROWBLK, C), lambda i: (i, 0)),
        out_shape=jax.ShapeDtypeStruct((ROWS, C), jnp.float32),
    )(x, w_t, b.reshape(1, C))


def _lane_maps():
    sm_cols = np.zeros((G * P, 128), np.float32)
    for g in range(G):
        for p in range(P):
            sm_cols[p * G + g, g * LANE + p] = 1.0
    seg = np.zeros((128, 128), np.float32)
    for g in range(G):
        for p in range(P):
            for q in range(P):
                seg[g * LANE + p, g * LANE + q] = 1.0
    offx = np.zeros((2 * P, 128), np.float32)
    offy = np.zeros((2 * P, 128), np.float32)
    for c in range(NCORN):
        for p in range(P):
            offx[p * 2 + 0, c * LANE + p] = float(W)
            offy[p * 2 + 1, c * LANE + p] = float(H)
    return sm_cols, seg, offx, offy


_SM_COLS, _SEG, _OFFX, _OFFY = _lane_maps()


@jax.jit
def kernel(feats, anchor_points, W_value, b_value, W_weights, b_weights,
           W_offset, b_offset, W_out, b_out):
    x = feats.reshape(BS, C, HW).transpose(0, 2, 1).reshape(ROWS, C)

    axy = (anchor_points.reshape(ROWS, 2) * jnp.float32(W)
           - jnp.float32(0.5))

    sm_cols = jnp.asarray(_SM_COLS)
    wsm = W_weights.T @ sm_cols
    bsm = (b_weights @ sm_cols).reshape(1, 128)
    wx = W_offset.T @ jnp.asarray(_OFFX)
    wy = W_offset.T @ jnp.asarray(_OFFY)
    bxy = jnp.stack([b_offset @ jnp.asarray(_OFFX),
                     b_offset @ jnp.asarray(_OFFY)])
    v, idx, coef = _prep(x, W_value.T, b_value.reshape(1, C),
                         wsm, bsm, jnp.asarray(_SEG), wx, wy, bxy, axy)
    out_pre = _sc_aggregate_fn()(v, idx, coef)
    out = _out_proj(out_pre, W_out.T, b_out)
    return out.reshape(BS, HW, C).transpose(0, 2, 1).reshape(BS, C, H, W)


# speedup vs baseline: 1.2808x; 1.0997x over previous
import functools

import jax
import jax.numpy as jnp
import numpy as np
from jax import lax
from jax.experimental import pallas as pl
from jax.experimental.pallas import tpu as pltpu
from jax.experimental.pallas import tpu_sc as plsc

C = 384
G = 8
P = 9
NCORN = 4
BS = 4
H = 32
W = 32
HW = H * W
ROWS = BS * HW
LANE = 16
NSLAB = C // LANE
QTR = HW // 4
NTASK = BS * NSLAB * 4
NWORK = 32
NT = NTASK // NWORK

_ROWBLK = 512


_NT = (((1,), (1,)), ((), ()))


def _prep_body(x_ref, wv_ref, bv_ref, wsm_ref, bsm_ref, seg_ref,
               wx_ref, wy_ref, bxy_ref, axy_ref,
               v_ref, idx_ref, coef_ref):
    x = x_ref[...]
    f32 = jnp.float32
    v_ref[...] = (jnp.dot(x, wv_ref[...], preferred_element_type=f32)
                  + bv_ref[...])
    e = jnp.exp(jnp.dot(x, wsm_ref[...], preferred_element_type=f32)
                + bsm_ref[...])
    z = jnp.dot(e, seg_ref[...], preferred_element_type=f32)
    sm = e / z
    xp = (jnp.dot(x, wx_ref[...], preferred_element_type=f32)
          + bxy_ref[0:1, :] + axy_ref[:, 0:1])
    yp = (jnp.dot(x, wy_ref[...], preferred_element_type=f32)
          + bxy_ref[1:2, :] + axy_ref[:, 1:2])
    lane = lax.broadcasted_iota(jnp.int32, xp.shape, 1)
    cidx = lane >> 4
    is_x1 = (cidx & 1) == 1
    is_y1 = cidx >= 2
    one = jnp.float32(1.0)
    xf = jnp.floor(xp)
    dx = xp - xf
    xc = xf + jnp.where(is_x1, one, 0.0)
    wxc = jnp.where(is_x1, dx, one - dx)
    yf = jnp.floor(yp)
    dy = yp - yf
    yc = yf + jnp.where(is_y1, one, 0.0)
    wyc = jnp.where(is_y1, dy, one - dy)
    valid = ((xc >= 0) & (xc <= W - 1) & (yc >= 0) & (yc <= H - 1))
    bw = wxc * wyc * jnp.where(valid, one, 0.0)
    xi = jnp.clip(xc, 0, W - 1).astype(jnp.int32)
    yi = jnp.clip(yc, 0, H - 1).astype(jnp.int32)
    idx = yi * W + xi
    idx_ref[...] = idx[:, 0:64]
    bw64 = bw[:, 0:64]
    parts = []
    for g in range(G):
        smg = sm[:, g * LANE:(g + 1) * LANE]
        smrep = jnp.concatenate([smg, smg, smg, smg], axis=1)
        parts.append(bw64 * smrep)
    coef_ref[...] = jnp.concatenate(parts, axis=1)


def _prep(x, wv, bv, wsm, bsm, seg, wx, wy, bxy, axy):
    nblk = ROWS // _ROWBLK
    row_spec = lambda nc: pl.BlockSpec((_ROWBLK, nc), lambda i: (i, 0))
    full = lambda a: pl.BlockSpec(a.shape, lambda i: (0,) * a.ndim)
    return pl.pallas_call(
        _prep_body,
        grid=(nblk,),
        in_specs=[row_spec(C), full(wv), full(bv), full(wsm), full(bsm),
                  full(seg), full(wx), full(wy), full(bxy), row_spec(2)],
        out_specs=[row_spec(C), row_spec(64), row_spec(G * 64)],
        out_shape=[
            jax.ShapeDtypeStruct((ROWS, C), jnp.float32),
            jax.ShapeDtypeStruct((ROWS, 64), jnp.int32),
            jax.ShapeDtypeStruct((ROWS, G * 64), jnp.float32),
        ],
    )(x, wv, bv, wsm, bsm, seg, wx, wy, bxy, axy)


def _sc_body(v_hbm, idx_hbm, coef_hbm, out_hbm,
             vslab0, vslab1, idx0, idx1, cf0, cf1, ov0, ov1,
             sem_a, sem_b, sem_o):
    wid = lax.axis_index("s") * 2 + lax.axis_index("c")
    vbufs = [vslab0, vslab1]
    ibufs = [idx0, idx1]
    cbufs = [cf0, cf1]
    obufs = [ov0, ov1]
    sems = [sem_a, sem_b]

    def params(r):
        pair = wid * (NT // 4) + r // 4
        qtr = r % 4
        b = pair // NSLAB
        slab = pair % NSLAB
        grp = slab // (NSLAB // G)
        row0 = b * HW + qtr * QTR
        col0 = slab * LANE
        return b, row0, col0, grp

    def start(r):
        b, row0, col0, grp = params(r)
        hs = []
        if r % 4 == 0:
            hs.append(pltpu.async_copy(
                v_hbm.at[pl.ds(b * HW, HW), pl.ds(col0, LANE)],
                vbufs[(r // 4) % 2], sems[r % 2]))
        hs.append(pltpu.async_copy(idx_hbm.at[pl.ds(row0, QTR)],
                                   ibufs[r % 2], sems[r % 2]))
        hs.append(pltpu.async_copy(
            coef_hbm.at[pl.ds(row0, QTR), pl.ds(grp * 64, 64)],
            cbufs[r % 2], sems[r % 2]))
        return hs

    hcur = start(0)
    outh = []
    for r in range(NT):
        hnext = start(r + 1) if r + 1 < NT else []
        if r >= 2:
            outh[r - 2].wait()
        for hh in hcur:
            hh.wait()
        vslab = vbufs[(r // 4) % 2]
        idx_v = ibufs[r % 2]
        coef_v = cbufs[r % 2]
        out_v = obufs[r % 2]

        @plsc.parallel_loop(0, QTR, unroll=4)
        def loc(i):
            idxr = [idx_v[i, pl.ds(c * LANE, LANE)] for c in range(NCORN)]
            cfr = [coef_v[i, pl.ds(c * LANE, LANE)] for c in range(NCORN)]
            acc = jnp.zeros((LANE,), jnp.float32)
            for pp in range(P):
                for c in range(NCORN):
                    rid = idxr[c][pp]
                    cfv = cfr[c][pp]
                    vals = vslab[rid]
                    acc = acc + vals * cfv
            out_v[i] = acc

        _, row0, col0, _ = params(r)
        outh.append(pltpu.async_copy(
            out_v, out_hbm.at[pl.ds(row0, QTR), pl.ds(col0, LANE)], sem_o))
        hcur = hnext
    outh[NT - 2].wait()
    outh[NT - 1].wait()


@functools.lru_cache(maxsize=1)
def _sc_aggregate_fn():
    return pl.kernel(
        _sc_body,
        out_type=jax.ShapeDtypeStruct((ROWS, C), jnp.float32),
        mesh=plsc.VectorSubcoreMesh(core_axis_name="c",
                                    subcore_axis_name="s"),
        compiler_params=pltpu.CompilerParams(use_tc_tiling_on_sc=False,
                                             needs_layout_passes=False),
        scratch_types=[
            pltpu.VMEM((HW, LANE), jnp.float32),
            pltpu.VMEM((HW, LANE), jnp.float32),
            pltpu.VMEM((QTR, 64), jnp.int32),
            pltpu.VMEM((QTR, 64), jnp.int32),
            pltpu.VMEM((QTR, 64), jnp.float32),
            pltpu.VMEM((QTR, 64), jnp.float32),
            pltpu.VMEM((QTR, LANE), jnp.float32),
            pltpu.VMEM((QTR, LANE), jnp.float32),
            pltpu.SemaphoreType.DMA,
            pltpu.SemaphoreType.DMA,
            pltpu.SemaphoreType.DMA,
        ],
    )


def _out_body(x_ref, w_ref, b_ref, o_ref):
    o_ref[...] = (jnp.dot(x_ref[...], w_ref[...],
                          preferred_element_type=jnp.float32) + b_ref[...])


def _out_proj(x, w_t, b):
    nblk = ROWS // _ROWBLK
    return pl.pallas_call(
        _out_body,
        grid=(nblk,),
        in_specs=[pl.BlockSpec((_ROWBLK, C), lambda i: (i, 0)),
                  pl.BlockSpec((C, C), lambda i: (0, 0)),
                  pl.BlockSpec((1, C), lambda i: (0, 0))],
        out_specs=pl.BlockSpec((_ROWBLK, C), lambda i: (i, 0)),
        out_shape=jax.ShapeDtypeStruct((ROWS, C), jnp.float32),
    )(x, w_t, b.reshape(1, C))


def _lane_maps():
    sm_cols = np.zeros((G * P, 128), np.float32)
    for g in range(G):
        for p in range(P):
            sm_cols[p * G + g, g * LANE + p] = 1.0
    seg = np.zeros((128, 128), np.float32)
    for g in range(G):
        for p in range(P):
            for q in range(P):
                seg[g * LANE + p, g * LANE + q] = 1.0
    offx = np.zeros((2 * P, 128), np.float32)
    offy = np.zeros((2 * P, 128), np.float32)
    for c in range(NCORN):
        for p in range(P):
            offx[p * 2 + 0, c * LANE + p] = float(W)
            offy[p * 2 + 1, c * LANE + p] = float(H)
    return sm_cols, seg, offx, offy


_SM_COLS, _SEG, _OFFX, _OFFY = _lane_maps()


@jax.jit
def kernel(feats, anchor_points, W_value, b_value, W_weights, b_weights,
           W_offset, b_offset, W_out, b_out):
    x = feats.reshape(BS, C, HW).transpose(0, 2, 1).reshape(ROWS, C)

    axy = (anchor_points.reshape(ROWS, 2) * jnp.float32(W)
           - jnp.float32(0.5))

    sm_cols = jnp.asarray(_SM_COLS)
    wsm = W_weights.T @ sm_cols
    bsm = (b_weights @ sm_cols).reshape(1, 128)
    wx = W_offset.T @ jnp.asarray(_OFFX)
    wy = W_offset.T @ jnp.asarray(_OFFY)
    bxy = jnp.stack([b_offset @ jnp.asarray(_OFFX),
                     b_offset @ jnp.asarray(_OFFY)])
    v, idx, coef = _prep(x, W_value.T, b_value.reshape(1, C),
                         wsm, bsm, jnp.asarray(_SEG), wx, wy, bxy, axy)
    out_pre = _sc_aggregate_fn()(v, idx, coef)
    out = _out_proj(out_pre, W_out.T, b_out)
    return out.reshape(BS, HW, C).transpose(0, 2, 1).reshape(BS, C, H, W)
